# baseline (speedup 1.0000x reference)
import jax
import jax.numpy as jnp
from jax.experimental import pallas as pl
from jax.experimental.pallas import tpu as pltpu

N_BLK = 1000


def _combine_body(msg_ref, ef_ref, adj_ref, deg_ref, inp_ref, out_ref):
    deg = deg_ref[...]
    aggr = ef_ref[...] / adj_ref[...]
    out_ref[...] = msg_ref[...] * deg + (1.0 - aggr) * inp_ref[...]


def _combine(msg, ef_sum, adj_rowwise, degree, inp):
    n, d = inp.shape
    grid = (n // N_BLK,)
    bs1 = pl.BlockSpec((N_BLK, 1), lambda i: (i, 0))
    bsd = pl.BlockSpec((N_BLK, d), lambda i: (i, 0))
    return pl.pallas_call(
        _combine_body,
        grid=grid,
        in_specs=[bsd, bs1, bs1, bs1, bsd],
        out_specs=bsd,
        out_shape=jax.ShapeDtypeStruct((n, d), jnp.float32),
    )(msg, ef_sum, adj_rowwise, degree[:, None], inp)


def _leaky(x):
    return jnp.where(x >= 0, x, 0.2 * x)


def kernel(input, adj, edge_factor, edges, adj_sparse_sum_rowwise, degree,
           iftrain, W2mini, b2mini, att_bias, Wf1, bf1, Wf2, bf2):
    Nn = input.shape[0]
    src = edges[0]
    dst = edges[1]
    h = input @ W2mini.T + b2mini
    h_src = jnp.take(h, src, axis=0)
    h_dst = jnp.take(h, dst, axis=0)
    h_diff = jnp.abs(h_dst - h_src)

    AH = W2mini.shape[0]
    A = Wf1[:, :AH]
    B = Wf1[:, AH:2 * AH]
    C = Wf1[:, 2 * AH:]
    z = h_src @ A.T + h_dst @ B.T + h_diff @ C.T + bf1
    cshift = (att_bias @ (A + B).T)[0]
    fc0 = jax.nn.sigmoid(_leaky(z) @ Wf2.T + bf2)[:, 0]
    fc1 = jax.nn.sigmoid(_leaky(z + cshift) @ Wf2.T + bf2)[:, 0]

    row_sum = jnp.zeros((Nn,), jnp.float32).at[src].add(fc0)
    fr = row_sum / adj_sparse_sum_rowwise[:, 0]
    nef = jnp.take(fr, src) * jnp.take(fr, dst) * fc1

    ef_sum = jnp.zeros((Nn, 1), jnp.float32).at[src].add(nef[:, None])
    scaled = input * degree[:, None]
    msg = jnp.zeros((Nn, input.shape[1]), jnp.float32).at[src].add(
        nef[:, None] * jnp.take(scaled, dst, axis=0))
    final_h = _combine(msg, ef_sum, adj_sparse_sum_rowwise, degree, input)
    return (final_h, nef)


# speedup vs baseline: 10.1778x; 10.1778x over previous
import functools

import jax
import jax.numpy as jnp
from jax import lax
from jax.experimental import pallas as pl
from jax.experimental.pallas import tpu as pltpu
from jax.experimental.pallas import tpu_sc as plsc

N = 10000
E = 320000
D = 128
AH = 16
NC = 2
NS = 16
NW = NC * NS
CH = 128
NCHUNK = E // CH
CPT = (NCHUNK + NW - 1) // NW
RPT = 624
RTAIL = N - NS * RPT

N_BLK = 1000


def _leaky(x):
    return jnp.where(x >= 0, x, 0.2 * x)


def _sigmoid(x):
    return 1.0 / (1.0 + jnp.exp(-x))


_SPLAT_DNUMS = lax.GatherDimensionNumbers(
    offset_dims=(), collapsed_slice_dims=(0,), start_index_map=(0,))


def _splat(vec, k):
    idx = jnp.full((16, 1), k, jnp.int32)
    return lax.gather(vec, idx, _SPLAT_DNUMS, (1,),
                      mode=lax.GatherScatterMode.PROMISE_IN_BOUNDS)


def _prep_body(x_ref, deg_ref, w2_ref, a_ref, b_ref, bf1_ref, b2_ref,
               tab_ref, scaled_ref):
    x = x_ref[...]
    h = lax.dot_general(x, w2_ref[...], (((1,), (1,)), ((), ())),
                        preferred_element_type=jnp.float32) + b2_ref[0:1, :]
    p = lax.dot_general(h, a_ref[...], (((1,), (1,)), ((), ())),
                        preferred_element_type=jnp.float32)
    q = lax.dot_general(h, b_ref[...], (((1,), (1,)), ((), ())),
                        preferred_element_type=jnp.float32) + bf1_ref[0:1, :]
    tab_ref[:, 0:AH] = h
    tab_ref[:, AH:2 * AH] = p
    tab_ref[:, 2 * AH:3 * AH] = q
    scaled_ref[...] = x * deg_ref[...]


def _prep(inp, degree, W2mini, A, B, bf1, b2mini):
    grid = (N // N_BLK,)
    return pl.pallas_call(
        _prep_body,
        grid=grid,
        in_specs=[
            pl.BlockSpec((N_BLK, D), lambda i: (i, 0)),
            pl.BlockSpec((N_BLK, 1), lambda i: (i, 0)),
            pl.BlockSpec((AH, D), lambda i: (0, 0)),
            pl.BlockSpec((AH, AH), lambda i: (0, 0)),
            pl.BlockSpec((AH, AH), lambda i: (0, 0)),
            pl.BlockSpec((1, AH), lambda i: (0, 0)),
            pl.BlockSpec((1, AH), lambda i: (0, 0)),
        ],
        out_specs=[
            pl.BlockSpec((N_BLK, 3 * AH), lambda i: (i, 0)),
            pl.BlockSpec((N_BLK, D), lambda i: (i, 0)),
        ],
        out_shape=[
            jax.ShapeDtypeStruct((N, 3 * AH), jnp.float32),
            jax.ShapeDtypeStruct((N, D), jnp.float32),
        ],
    )(inp, degree[:, None], W2mini, A, B, bf1[None, :], b2mini[None, :])


def _passa_body(tab_h, src_h, dst_h, wsm_h, z1_h,
                fc1o_h, rsp_h,
                srows_v, drows_v, sidx_v, didx_v, fc0_v, fc1_v,
                sbuf_v, dbuf_v, wtmp_v, rsb_v, rs_s, gsem, gsem2):
    c = lax.axis_index("c")
    s = lax.axis_index("s")
    wid = s * NC + c

    pltpu.sync_copy(wsm_h, wtmp_v)
    @pl.when(s == 0)
    def _():
        pltpu.sync_copy(z1_h, rs_s)
    plsc.subcore_barrier()

    iota = lax.iota(jnp.int32, 16)

    def chunk_body(i, carry):
        cid = i * NW + wid

        @pl.when(cid < NCHUNK)
        def _():
            base = cid * CH
            pltpu.sync_copy(src_h.at[pl.ds(base, CH)], sidx_v)
            pltpu.sync_copy(dst_h.at[pl.ds(base, CH)], didx_v)
            pltpu.async_copy(tab_h.at[sidx_v], srows_v, gsem).wait()
            pltpu.async_copy(tab_h.at[didx_v], drows_v, gsem2).wait()

            def group(g, carry2):
                for e in range(16):
                    row = g * 16 + e
                    hs = srows_v[row, pl.ds(0, AH)]
                    ps = srows_v[row, pl.ds(AH, AH)]
                    hd = drows_v[row, pl.ds(0, AH)]
                    qd = drows_v[row, pl.ds(2 * AH, AH)]
                    sbuf_v[e, pl.ds(0, 16)] = ps + qd
                    dbuf_v[e, pl.ds(0, 16)] = jnp.abs(hd - hs)
                dT = [plsc.load_gather(
                          dbuf_v, [iota, jnp.full((16,), k, jnp.int32)])
                      for k in range(AH)]
                w2v = wtmp_v[16, :]
                csv = wtmp_v[17, :]
                bf2 = _splat(wtmp_v[18, :], 0)
                y0 = jnp.zeros((16,), jnp.float32)
                y1 = jnp.zeros((16,), jnp.float32)
                for j in range(AH):
                    acc = plsc.load_gather(
                        sbuf_v, [iota, jnp.full((16,), j, jnp.int32)])
                    crow = wtmp_v[j, :]
                    for k in range(AH):
                        acc = acc + _splat(crow, k) * dT[k]
                    w2j = _splat(w2v, j)
                    y0 = y0 + w2j * _leaky(acc)
                    y1 = y1 + w2j * _leaky(acc + _splat(csv, j))
                fc0_v[pl.ds(g * 16, 16)] = _sigmoid(y0 + bf2)
                fc1_v[pl.ds(g * 16, 16)] = _sigmoid(y1 + bf2)
                return carry2

            lax.fori_loop(0, CH // 16, group, 0)
            pltpu.sync_copy(fc1_v, fc1o_h.at[pl.ds(base, CH)])
            pltpu.sync_copy(fc0_v, rs_s.at[sidx_v], add=True)
        return carry

    lax.fori_loop(0, CPT, chunk_body, 0)
    plsc.subcore_barrier()
    @pl.when(s == 0)
    def _():
        pltpu.sync_copy(rs_s, rsb_v)
        pltpu.sync_copy(rsb_v, rsp_h.at[pl.ds(c * N, N)])


_passa = functools.partial(
    pl.kernel,
    _passa_body,
    out_type=[
        jax.ShapeDtypeStruct((E,), jnp.float32),
        jax.ShapeDtypeStruct((NC * N,), jnp.float32),
    ],
    mesh=plsc.VectorSubcoreMesh(core_axis_name="c", subcore_axis_name="s"),
    scratch_types=[
        pltpu.VMEM((CH, 3 * AH), jnp.float32),
        pltpu.VMEM((CH, 3 * AH), jnp.float32),
        pltpu.VMEM((CH,), jnp.int32),
        pltpu.VMEM((CH,), jnp.int32),
        pltpu.VMEM((CH,), jnp.float32),
        pltpu.VMEM((CH,), jnp.float32),
        pltpu.VMEM((16, 17), jnp.float32),
        pltpu.VMEM((16, 17), jnp.float32),
        pltpu.VMEM((19, 16), jnp.float32),
        pltpu.VMEM((N,), jnp.float32),
        pltpu.VMEM_SHARED((N,), jnp.float32),
        pltpu.SemaphoreType.DMA,
        pltpu.SemaphoreType.DMA,
    ],
    compiler_params=pltpu.CompilerParams(needs_layout_passes=False,
                                         use_tc_tiling_on_sc=False),
)()


def _fr_body(rsp_ref, adj_ref, fr_ref):
    fr_ref[...] = (rsp_ref[0] + rsp_ref[1]) / adj_ref[...]


def _fr_reduce(rsp, adj):
    r = rsp.reshape(NC, 8, N // 8)
    a = adj.reshape(8, N // 8)
    out = pl.pallas_call(
        _fr_body,
        out_shape=jax.ShapeDtypeStruct((8, N // 8), jnp.float32),
    )(r, a)
    return out.reshape(N)


def _passb_body(fr_h, fc1_h, src_h, dst_h, scaled_h, z2_h, z1_h,
                nef_h, msgp_h, efp_h,
                fr_v, rows_v, sidx_v, didx_v, fc1_v, nef_v,
                msg_s, ef_s, gsem):
    c = lax.axis_index("c")
    s = lax.axis_index("s")
    wid = s * NC + c

    pltpu.sync_copy(z2_h.at[pl.ds(s * RPT, RPT)], msg_s.at[pl.ds(s * RPT, RPT)])
    @pl.when(s == 0)
    def _():
        pltpu.sync_copy(z2_h.at[pl.ds(NS * RPT, RTAIL)],
                        msg_s.at[pl.ds(NS * RPT, RTAIL)])
        pltpu.sync_copy(z1_h, ef_s)
    pltpu.sync_copy(fr_h, fr_v)
    plsc.subcore_barrier()

    def chunk_body(i, carry):
        cid = i * NW + wid

        @pl.when(cid < NCHUNK)
        def _():
            base = cid * CH
            pltpu.sync_copy(src_h.at[pl.ds(base, CH)], sidx_v)
            pltpu.sync_copy(dst_h.at[pl.ds(base, CH)], didx_v)
            pltpu.sync_copy(fc1_h.at[pl.ds(base, CH)], fc1_v)
            pltpu.async_copy(scaled_h.at[didx_v], rows_v, gsem).wait()
            for g in range(CH // 16):
                sg = sidx_v[pl.ds(g * 16, 16)]
                dg = didx_v[pl.ds(g * 16, 16)]
                frs = plsc.load_gather(fr_v, [sg])
                frd = plsc.load_gather(fr_v, [dg])
                nef_v[pl.ds(g * 16, 16)] = frs * frd * fc1_v[pl.ds(g * 16, 16)]

            def scale_rows(g, carry2):
                nef16 = nef_v[pl.ds(g * 16, 16)]
                for el in range(16):
                    e = g * 16 + el
                    sc = _splat(nef16, el)
                    for j in range(D // 16):
                        rows_v[e, pl.ds(j * 16, 16)] = (
                            rows_v[e, pl.ds(j * 16, 16)] * sc)
                return carry2

            lax.fori_loop(0, CH // 16, scale_rows, 0)
            pltpu.sync_copy(nef_v, nef_h.at[pl.ds(base, CH)])
            pltpu.sync_copy(nef_v, ef_s.at[sidx_v], add=True)
            pltpu.sync_copy(rows_v, msg_s.at[sidx_v], add=True)
        return carry

    lax.fori_loop(0, CPT, chunk_body, 0)
    plsc.subcore_barrier()
    pltpu.sync_copy(msg_s.at[pl.ds(s * RPT, RPT)], msgp_h.at[c, pl.ds(s * RPT, RPT)])
    @pl.when(s == 0)
    def _():
        pltpu.sync_copy(msg_s.at[pl.ds(NS * RPT, RTAIL)],
                        msgp_h.at[c, pl.ds(NS * RPT, RTAIL)])
        pltpu.sync_copy(ef_s, fr_v)
        pltpu.sync_copy(fr_v, efp_h.at[pl.ds(c * N, N)])


_passb = functools.partial(
    pl.kernel,
    _passb_body,
    out_type=[
        jax.ShapeDtypeStruct((E,), jnp.float32),
        jax.ShapeDtypeStruct((NC, N, D), jnp.float32),
        jax.ShapeDtypeStruct((NC * N,), jnp.float32),
    ],
    mesh=plsc.VectorSubcoreMesh(core_axis_name="c", subcore_axis_name="s"),
    scratch_types=[
        pltpu.VMEM((N,), jnp.float32),
        pltpu.VMEM((CH, D), jnp.float32),
        pltpu.VMEM((CH,), jnp.int32),
        pltpu.VMEM((CH,), jnp.int32),
        pltpu.VMEM((CH,), jnp.float32),
        pltpu.VMEM((CH,), jnp.float32),
        pltpu.VMEM_SHARED((N, D), jnp.float32),
        pltpu.VMEM_SHARED((N,), jnp.float32),
        pltpu.SemaphoreType.DMA,
    ],
    compiler_params=pltpu.CompilerParams(needs_layout_passes=False),
)()


def _combine_body(msgp_ref, efp_ref, adj_ref, deg_ref, inp_ref, out_ref):
    deg = deg_ref[...]
    msg = msgp_ref[0] + msgp_ref[1]
    ef = efp_ref[0] + efp_ref[1]
    aggr = ef / adj_ref[...]
    out_ref[...] = msg * deg + (1.0 - aggr) * inp_ref[...]


def _combine(msgp, efp, adj_rowwise, degree, inp):
    n, d = inp.shape
    grid = (n // N_BLK,)
    bs1 = pl.BlockSpec((N_BLK, 1), lambda i: (i, 0))
    bsd = pl.BlockSpec((N_BLK, d), lambda i: (i, 0))
    bs2 = pl.BlockSpec((NC, N_BLK, 1), lambda i: (0, i, 0))
    bs2d = pl.BlockSpec((NC, N_BLK, d), lambda i: (0, i, 0))
    return pl.pallas_call(
        _combine_body,
        grid=grid,
        in_specs=[bs2d, bs2, bs1, bs1, bsd],
        out_specs=bsd,
        out_shape=jax.ShapeDtypeStruct((n, d), jnp.float32),
    )(msgp, efp[:, :, None], adj_rowwise, degree[:, None], inp)


def kernel(input, adj, edge_factor, edges, adj_sparse_sum_rowwise, degree,
           iftrain, W2mini, b2mini, att_bias, Wf1, bf1, Wf2, bf2):
    src = edges[0].astype(jnp.int32)
    dst = edges[1].astype(jnp.int32)
    A = Wf1[:, :AH]
    B = Wf1[:, AH:2 * AH]
    C = Wf1[:, 2 * AH:]
    cshift = (att_bias @ (A + B).T)[0]

    tab, scaled = _prep(input, degree, W2mini, A, B, bf1, b2mini)

    wsm = jnp.concatenate([
        C,
        Wf2[0][None, :],
        cshift[None, :],
        jnp.full((1, AH), bf2[0], jnp.float32),
    ], axis=0)

    z1 = jnp.zeros((N,), jnp.float32)
    fc1, rsp = _passa(tab, src, dst, wsm, z1)
    fr = _fr_reduce(rsp, adj_sparse_sum_rowwise[:, 0])

    z2 = jnp.zeros((N, D), jnp.float32)
    nef, msgp, efp = _passb(fr, fc1, src, dst, scaled, z2, z1)
    final_h = _combine(msgp, efp.reshape(NC, N), adj_sparse_sum_rowwise,
                       degree, input)
    return (final_h, nef)
